# Initial kernel scaffold; baseline (speedup 1.0000x reference)
import functools

import jax
import jax.numpy as jnp
from jax import lax
from jax.experimental import pallas as pl
from jax.experimental.pallas import tpu as pltpu
from jax.experimental.pallas import tpu_sc as plsc

_NC = 2
_NS = 16
_NW = _NC * _NS
_CH = 80
_BN = 1280


def _mm3(x, w0, b0, w1, b1, w2, b2):
    n, d = x.shape
    h = w0.shape[1]

    def body(x_ref, w0_ref, b0_ref, w1_ref, b1_ref, w2_ref, b2_ref,
             o0_ref, o1_ref, o2_ref):
        xv = x_ref[...]
        o0_ref[...] = jnp.dot(xv, w0_ref[...], preferred_element_type=jnp.float32) + b0_ref[...]
        o1_ref[...] = jnp.dot(xv, w1_ref[...], preferred_element_type=jnp.float32) + b1_ref[...]
        o2_ref[...] = jnp.dot(xv, w2_ref[...], preferred_element_type=jnp.float32) + b2_ref[...]

    wspec = pl.BlockSpec((d, h), lambda i: (0, 0))
    bspec = pl.BlockSpec((1, h), lambda i: (0, 0))
    ospec = pl.BlockSpec((_BN, h), lambda i: (i, 0))
    return pl.pallas_call(
        body,
        grid=(n // _BN,),
        in_specs=[pl.BlockSpec((_BN, d), lambda i: (i, 0)),
                  wspec, bspec, wspec, bspec, wspec, bspec],
        out_specs=[ospec, ospec, ospec],
        out_shape=[jax.ShapeDtypeStruct((n, h), jnp.float32)] * 3,
    )(x, w0, b0.reshape(1, -1), w1, b1.reshape(1, -1), w2, b2.reshape(1, -1))


def _mid(num, den, sk, b1, w0, b0, w1, bb1, w2, b2):
    npad = num.shape[1]
    d = num.shape[2]
    h = w0.shape[1]

    def body(num_ref, den_ref, sk_ref, b1_ref, w0_ref, b0_ref, w1_ref,
             bb1_ref, w2_ref, b2_ref, o0_ref, o1_ref, o2_ref):
        nm = num_ref[0] + num_ref[1]
        dn = jnp.sum(den_ref[...], axis=0)
        hv = jnp.maximum(nm / (dn[:, None] + 1e-16) + b1_ref[...] + sk_ref[...], 0.0)
        o0_ref[...] = jnp.dot(hv, w0_ref[...], preferred_element_type=jnp.float32) + b0_ref[...]
        o1_ref[...] = jnp.dot(hv, w1_ref[...], preferred_element_type=jnp.float32) + bb1_ref[...]
        o2_ref[...] = jnp.dot(hv, w2_ref[...], preferred_element_type=jnp.float32) + b2_ref[...]

    wspec = pl.BlockSpec((d, h), lambda i: (0, 0))
    bspec = pl.BlockSpec((1, h), lambda i: (0, 0))
    ospec = pl.BlockSpec((_BN, h), lambda i: (i, 0))
    return pl.pallas_call(
        body,
        grid=(npad // _BN,),
        in_specs=[pl.BlockSpec((_NC, _BN, d), lambda i: (0, i, 0)),
                  pl.BlockSpec((_NW, _BN), lambda i: (0, i)),
                  pl.BlockSpec((_BN, d), lambda i: (i, 0)),
                  bspec, wspec, bspec, wspec, bspec, wspec, bspec],
        out_specs=[ospec, ospec, ospec],
        out_shape=[jax.ShapeDtypeStruct((npad, h), jnp.float32)] * 3,
    )(num, den, sk, b1.reshape(1, -1), w0, b0.reshape(1, -1), w1,
      bb1.reshape(1, -1), w2, b2.reshape(1, -1))


def _final(num, den, sk, b2):
    npad = num.shape[1]
    d = num.shape[2]

    def body(num_ref, den_ref, sk_ref, b2_ref, o_ref):
        nm = num_ref[0] + num_ref[1]
        dn = jnp.sum(den_ref[...], axis=0)
        o_ref[...] = nm / (dn[:, None] + 1e-16) + b2_ref[...] + sk_ref[...]

    return pl.pallas_call(
        body,
        grid=(npad // _BN,),
        in_specs=[pl.BlockSpec((_NC, _BN, d), lambda i: (0, i, 0)),
                  pl.BlockSpec((_NW, _BN), lambda i: (0, i)),
                  pl.BlockSpec((_BN, d), lambda i: (i, 0)),
                  pl.BlockSpec((1, d), lambda i: (0, 0))],
        out_specs=pl.BlockSpec((_BN, d), lambda i: (i, 0)),
        out_shape=jax.ShapeDtypeStruct((npad, d), jnp.float32),
    )(num, den, sk, b2.reshape(1, -1))


def _edge_agg(xl, xr, ei, att):
    npad, d = xl.shape
    e = ei.shape[1]
    ew = e // _NW
    nch = ew // _CH
    rps = npad // _NS
    nj = d // 16

    mesh = plsc.VectorSubcoreMesh(core_axis_name="c", subcore_axis_name="s",
                                  num_cores=_NC, num_subcores=_NS)

    @functools.partial(
        pl.kernel,
        out_type=[jax.ShapeDtypeStruct((_NC, npad, d), jnp.float32),
                  jax.ShapeDtypeStruct((_NW, npad), jnp.float32)],
        mesh=mesh,
        scratch_types=[
            pltpu.VMEM((_CH,), jnp.int32),
            pltpu.VMEM((_CH,), jnp.int32),
            pltpu.VMEM((_CH, d), jnp.float32),
            pltpu.VMEM((_CH, d), jnp.float32),
            pltpu.VMEM((d,), jnp.float32),
            pltpu.VMEM((npad,), jnp.float32),
            pltpu.VMEM_SHARED((npad, d), jnp.float32),
            pltpu.SemaphoreType.DMA,
            pltpu.SemaphoreType.DMA,
        ],
    )
    def k(xl_h, xr_h, ei_h, att_h, num_h, den_h,
          srcv, dstv, xls, xrd, attv, denv, numsh, sem1, sem2):
        cid = lax.axis_index("c")
        sid = lax.axis_index("s")
        wid = cid * _NS + sid
        zero16 = jnp.zeros((16,), jnp.float32)

        def zrow(i, _):
            for j in range(nj):
                xls[i, pl.ds(16 * j, 16)] = zero16
            return 0
        lax.fori_loop(0, _CH, zrow, 0)

        def zden(i, _):
            denv[pl.ds(i * 16, 16)] = zero16
            return 0
        lax.fori_loop(0, npad // 16, zden, 0)

        def znum(i, _):
            pltpu.sync_copy(xls, numsh.at[pl.ds(sid * rps + i * _CH, _CH)])
            return 0
        lax.fori_loop(0, rps // _CH, znum, 0)

        pltpu.sync_copy(att_h, attv)
        plsc.subcore_barrier()

        attr = [attv[pl.ds(16 * j, 16)] for j in range(nj)]
        lane0 = lax.iota(jnp.int32, 16) == 0
        base_e = wid * ew

        def chunk(c, _):
            off = base_e + c * _CH
            pltpu.sync_copy(ei_h.at[0, pl.ds(off, _CH)], srcv)
            pltpu.sync_copy(ei_h.at[1, pl.ds(off, _CH)], dstv)
            pltpu.async_copy(xl_h.at[srcv], xls, sem1).wait()
            pltpu.async_copy(xr_h.at[dstv], xrd, sem2).wait()

            def edge(ee, _):
                lvs = []
                acc = zero16
                for j in range(nj):
                    lv = xls[ee, pl.ds(16 * j, 16)]
                    rv = xrd[ee, pl.ds(16 * j, 16)]
                    lvs.append(lv)
                    m = lv + rv
                    m = jnp.maximum(m, 0.2 * m)
                    acc = acc + m * attr[j]
                ex = jnp.exp(jnp.full((16,), jnp.sum(acc), jnp.float32))
                for j in range(nj):
                    xls[ee, pl.ds(16 * j, 16)] = ex * lvs[j]
                dsc = dstv[ee]
                plsc.addupdate_scatter(denv, [jnp.full((16,), dsc, jnp.int32)],
                                       ex, mask=lane0)
                return 0
            lax.fori_loop(0, _CH, edge, 0)

            pltpu.sync_copy(xls, numsh.at[dstv], add=True)
            return 0
        lax.fori_loop(0, nch, chunk, 0)

        plsc.subcore_barrier()
        pltpu.sync_copy(denv, den_h.at[wid])

        def wout(i, _):
            r0 = sid * rps + i * _CH
            pltpu.sync_copy(numsh.at[pl.ds(r0, _CH)], num_h.at[cid, pl.ds(r0, _CH)])
            return 0
        lax.fori_loop(0, rps // _CH, wout, 0)

    return k(xl, xr, ei, att)


def kernel(x, edge_index, edge_attr, Wl1, bl1, Wr1, br1, att1, bias1, Wlin1, blin1,
           Wl2, bl2, Wr2, br2, att2, bias2, Wlin2, blin2):
    n = x.shape[0]
    npad = ((n + _BN - 1) // _BN) * _BN
    xp = jnp.pad(x, ((0, npad - n), (0, 0)))

    xl1, xr1, sk1 = _mm3(xp, Wl1, bl1, Wr1, br1, Wlin1, blin1)
    num1, den1 = _edge_agg(xl1, xr1, edge_index, att1)
    xl2, xr2, sk2 = _mid(num1, den1, sk1, bias1, Wl2, bl2, Wr2, br2, Wlin2, blin2)
    num2, den2 = _edge_agg(xl2, xr2, edge_index, att2)
    outp = _final(num2, den2, sk2, bias2)
    return outp[:n]


# speedup vs baseline: 7.9830x; 7.9830x over previous
import functools

import jax
import jax.numpy as jnp
from jax import lax
from jax.experimental import pallas as pl
from jax.experimental.pallas import tpu as pltpu
from jax.experimental.pallas import tpu_sc as plsc

_NC = 2
_NS = 16
_NW = _NC * _NS
_CH = 80
_BN = 1280


def _mm3(x, w0, b0, w1, b1, w2, b2):
    n, d = x.shape
    h = w0.shape[1]

    def body(x_ref, w0_ref, b0_ref, w1_ref, b1_ref, w2_ref, b2_ref,
             o0_ref, o1_ref, o2_ref):
        xv = x_ref[...]
        o0_ref[...] = jnp.dot(xv, w0_ref[...], preferred_element_type=jnp.float32) + b0_ref[...]
        o1_ref[...] = jnp.dot(xv, w1_ref[...], preferred_element_type=jnp.float32) + b1_ref[...]
        o2_ref[...] = jnp.dot(xv, w2_ref[...], preferred_element_type=jnp.float32) + b2_ref[...]

    wspec = pl.BlockSpec((d, h), lambda i: (0, 0))
    bspec = pl.BlockSpec((1, h), lambda i: (0, 0))
    ospec = pl.BlockSpec((_BN, h), lambda i: (i, 0))
    return pl.pallas_call(
        body,
        grid=(n // _BN,),
        in_specs=[pl.BlockSpec((_BN, d), lambda i: (i, 0)),
                  wspec, bspec, wspec, bspec, wspec, bspec],
        out_specs=[ospec, ospec, ospec],
        out_shape=[jax.ShapeDtypeStruct((n, h), jnp.float32)] * 3,
    )(x, w0, b0.reshape(1, -1), w1, b1.reshape(1, -1), w2, b2.reshape(1, -1))


def _mid(num, den, sk, b1, w0, b0, w1, bb1, w2, b2):
    npad = num.shape[1]
    d = num.shape[2]
    h = w0.shape[1]

    def body(num_ref, den_ref, sk_ref, b1_ref, w0_ref, b0_ref, w1_ref,
             bb1_ref, w2_ref, b2_ref, o0_ref, o1_ref, o2_ref):
        nm = num_ref[0] + num_ref[1]
        dn = jnp.sum(den_ref[...], axis=0)
        hv = jnp.maximum(nm / (dn[:, None] + 1e-16) + b1_ref[...] + sk_ref[...], 0.0)
        o0_ref[...] = jnp.dot(hv, w0_ref[...], preferred_element_type=jnp.float32) + b0_ref[...]
        o1_ref[...] = jnp.dot(hv, w1_ref[...], preferred_element_type=jnp.float32) + bb1_ref[...]
        o2_ref[...] = jnp.dot(hv, w2_ref[...], preferred_element_type=jnp.float32) + b2_ref[...]

    wspec = pl.BlockSpec((d, h), lambda i: (0, 0))
    bspec = pl.BlockSpec((1, h), lambda i: (0, 0))
    ospec = pl.BlockSpec((_BN, h), lambda i: (i, 0))
    return pl.pallas_call(
        body,
        grid=(npad // _BN,),
        in_specs=[pl.BlockSpec((_NC, _BN, d), lambda i: (0, i, 0)),
                  pl.BlockSpec((_NW, _BN), lambda i: (0, i)),
                  pl.BlockSpec((_BN, d), lambda i: (i, 0)),
                  bspec, wspec, bspec, wspec, bspec, wspec, bspec],
        out_specs=[ospec, ospec, ospec],
        out_shape=[jax.ShapeDtypeStruct((npad, h), jnp.float32)] * 3,
    )(num, den, sk, b1.reshape(1, -1), w0, b0.reshape(1, -1), w1,
      bb1.reshape(1, -1), w2, b2.reshape(1, -1))


def _final(num, den, sk, b2):
    npad = num.shape[1]
    d = num.shape[2]

    def body(num_ref, den_ref, sk_ref, b2_ref, o_ref):
        nm = num_ref[0] + num_ref[1]
        dn = jnp.sum(den_ref[...], axis=0)
        o_ref[...] = nm / (dn[:, None] + 1e-16) + b2_ref[...] + sk_ref[...]

    return pl.pallas_call(
        body,
        grid=(npad // _BN,),
        in_specs=[pl.BlockSpec((_NC, _BN, d), lambda i: (0, i, 0)),
                  pl.BlockSpec((_NW, _BN), lambda i: (0, i)),
                  pl.BlockSpec((_BN, d), lambda i: (i, 0)),
                  pl.BlockSpec((1, d), lambda i: (0, 0))],
        out_specs=pl.BlockSpec((_BN, d), lambda i: (i, 0)),
        out_shape=jax.ShapeDtypeStruct((npad, d), jnp.float32),
    )(num, den, sk, b2.reshape(1, -1))


def _edge_agg(xl, xr, src, dst, att):
    npad, d = xl.shape
    e = src.shape[0]
    ew = e // _NW
    nch = ew // _CH
    rps = npad // _NS
    nj = d // 16

    mesh = plsc.VectorSubcoreMesh(core_axis_name="c", subcore_axis_name="s",
                                  num_cores=_NC, num_subcores=_NS)

    @functools.partial(
        pl.kernel,
        out_type=[jax.ShapeDtypeStruct((_NC, npad, d), jnp.float32),
                  jax.ShapeDtypeStruct((_NW * npad,), jnp.float32)],
        mesh=mesh,
        compiler_params=pltpu.CompilerParams(needs_layout_passes=False),
        scratch_types=[
            pltpu.VMEM((_CH,), jnp.int32),
            pltpu.VMEM((_CH,), jnp.int32),
            pltpu.VMEM((_CH, d), jnp.float32),
            pltpu.VMEM((_CH, d), jnp.float32),
            pltpu.VMEM((d,), jnp.float32),
            pltpu.VMEM((npad,), jnp.float32),
            pltpu.VMEM_SHARED((npad, d), jnp.float32),
            pltpu.SemaphoreType.DMA,
            pltpu.SemaphoreType.DMA,
        ],
    )
    def k(xl_h, xr_h, src_h, dst_h, att_h, num_h, den_h,
          srcv, dstv, xls, xrd, attv, denv, numsh, sem1, sem2):
        cid = lax.axis_index("c")
        sid = lax.axis_index("s")
        wid = cid * _NS + sid
        zero16 = jnp.zeros((16,), jnp.float32)

        def zrow(i, _):
            for j in range(nj):
                xls[i, pl.ds(16 * j, 16)] = zero16
            return 0
        lax.fori_loop(0, _CH, zrow, 0)

        def zden(i, _):
            denv[pl.ds(i * 16, 16)] = zero16
            return 0
        lax.fori_loop(0, npad // 16, zden, 0)

        def znum(i, _):
            pltpu.sync_copy(xls, numsh.at[pl.ds(sid * rps + i * _CH, _CH)])
            return 0
        lax.fori_loop(0, rps // _CH, znum, 0)

        pltpu.sync_copy(att_h, attv)
        plsc.subcore_barrier()

        attr = [attv[pl.ds(16 * j, 16)] for j in range(nj)]
        lane0 = lax.iota(jnp.int32, 16) == 0
        base_e = wid * ew

        def chunk(c, _):
            off = base_e + c * _CH
            pltpu.sync_copy(src_h.at[pl.ds(off, _CH)], srcv)
            pltpu.sync_copy(dst_h.at[pl.ds(off, _CH)], dstv)
            pltpu.async_copy(xl_h.at[srcv], xls, sem1).wait()
            pltpu.async_copy(xr_h.at[dstv], xrd, sem2).wait()

            def group(g, _):
                dvec = dstv[pl.ds(g * 16, 16)]
                for i in range(16):
                    ee = g * 16 + i
                    lvs = []
                    acc = zero16
                    for j in range(nj):
                        lv = xls[ee, pl.ds(16 * j, 16)]
                        rv = xrd[ee, pl.ds(16 * j, 16)]
                        lvs.append(lv)
                        m = lv + rv
                        m = jnp.maximum(m, 0.2 * m)
                        acc = acc + m * attr[j]
                    ex = jnp.exp(jnp.full((16,), jnp.sum(acc), jnp.float32))
                    for j in range(nj):
                        xls[ee, pl.ds(16 * j, 16)] = ex * lvs[j]
                    plsc.addupdate_scatter(denv,
                                           [jnp.full((16,), dvec[i], jnp.int32)],
                                           ex, mask=lane0)
                return 0
            lax.fori_loop(0, _CH // 16, group, 0)

            pltpu.sync_copy(xls, numsh.at[dstv], add=True)
            return 0
        lax.fori_loop(0, nch, chunk, 0)

        plsc.subcore_barrier()
        pltpu.sync_copy(denv, den_h.at[pl.ds(wid * npad, npad)])

        def wout(i, _):
            r0 = sid * rps + i * _CH
            pltpu.sync_copy(numsh.at[pl.ds(r0, _CH)], num_h.at[cid, pl.ds(r0, _CH)])
            return 0
        lax.fori_loop(0, rps // _CH, wout, 0)

    num, den_flat = k(xl, xr, src, dst, att)
    return num, den_flat.reshape(_NW, npad)


def kernel(x, edge_index, edge_attr, Wl1, bl1, Wr1, br1, att1, bias1, Wlin1, blin1,
           Wl2, bl2, Wr2, br2, att2, bias2, Wlin2, blin2):
    n = x.shape[0]
    npad = ((n + _BN - 1) // _BN) * _BN
    xp = jnp.pad(x, ((0, npad - n), (0, 0)))

    src = edge_index[0]
    dst = edge_index[1]
    xl1, xr1, sk1 = _mm3(xp, Wl1, bl1, Wr1, br1, Wlin1, blin1)
    num1, den1 = _edge_agg(xl1, xr1, src, dst, att1)
    xl2, xr2, sk2 = _mid(num1, den1, sk1, bias1, Wl2, bl2, Wr2, br2, Wlin2, blin2)
    num2, den2 = _edge_agg(xl2, xr2, src, dst, att2)
    outp = _final(num2, den2, sk2, bias2)
    return outp[:n]


# speedup vs baseline: 13.0828x; 1.6388x over previous
import functools

import jax
import jax.numpy as jnp
from jax import lax
from jax.experimental import pallas as pl
from jax.experimental.pallas import tpu as pltpu
from jax.experimental.pallas import tpu_sc as plsc

_NC = 2
_NS = 16
_NW = _NC * _NS
_CH = 64
_PRE = 4
_BN = 1280


def _mm3(x, w0, b0, w1, b1, w2, b2):
    n, d = x.shape
    h = w0.shape[1]

    def body(x_ref, w0_ref, b0_ref, w1_ref, b1_ref, w2_ref, b2_ref,
             o0_ref, o1_ref, o2_ref):
        xv = x_ref[...]
        o0_ref[...] = jnp.dot(xv, w0_ref[...], preferred_element_type=jnp.float32) + b0_ref[...]
        o1_ref[...] = jnp.dot(xv, w1_ref[...], preferred_element_type=jnp.float32) + b1_ref[...]
        o2_ref[...] = jnp.dot(xv, w2_ref[...], preferred_element_type=jnp.float32) + b2_ref[...]

    wspec = pl.BlockSpec((d, h), lambda i: (0, 0))
    bspec = pl.BlockSpec((1, h), lambda i: (0, 0))
    ospec = pl.BlockSpec((_BN, h), lambda i: (i, 0))
    return pl.pallas_call(
        body,
        grid=(n // _BN,),
        in_specs=[pl.BlockSpec((_BN, d), lambda i: (i, 0)),
                  wspec, bspec, wspec, bspec, wspec, bspec],
        out_specs=[ospec, ospec, ospec],
        out_shape=[jax.ShapeDtypeStruct((n, h), jnp.float32)] * 3,
    )(x, w0, b0.reshape(1, -1), w1, b1.reshape(1, -1), w2, b2.reshape(1, -1))


def _mid(num, den, sk, b1, w0, b0, w1, bb1, w2, b2):
    npad = num.shape[1]
    d = num.shape[2]
    h = w0.shape[1]

    def body(num_ref, den_ref, sk_ref, b1_ref, w0_ref, b0_ref, w1_ref,
             bb1_ref, w2_ref, b2_ref, o0_ref, o1_ref, o2_ref):
        nm = num_ref[0] + num_ref[1]
        dn = jnp.sum(den_ref[...], axis=0)
        hv = jnp.maximum(nm / (dn[:, None] + 1e-16) + b1_ref[...] + sk_ref[...], 0.0)
        o0_ref[...] = jnp.dot(hv, w0_ref[...], preferred_element_type=jnp.float32) + b0_ref[...]
        o1_ref[...] = jnp.dot(hv, w1_ref[...], preferred_element_type=jnp.float32) + bb1_ref[...]
        o2_ref[...] = jnp.dot(hv, w2_ref[...], preferred_element_type=jnp.float32) + b2_ref[...]

    wspec = pl.BlockSpec((d, h), lambda i: (0, 0))
    bspec = pl.BlockSpec((1, h), lambda i: (0, 0))
    ospec = pl.BlockSpec((_BN, h), lambda i: (i, 0))
    return pl.pallas_call(
        body,
        grid=(npad // _BN,),
        in_specs=[pl.BlockSpec((_NC, _BN, d), lambda i: (0, i, 0)),
                  pl.BlockSpec((_NW, _BN), lambda i: (0, i)),
                  pl.BlockSpec((_BN, d), lambda i: (i, 0)),
                  bspec, wspec, bspec, wspec, bspec, wspec, bspec],
        out_specs=[ospec, ospec, ospec],
        out_shape=[jax.ShapeDtypeStruct((npad, h), jnp.float32)] * 3,
    )(num, den, sk, b1.reshape(1, -1), w0, b0.reshape(1, -1), w1,
      bb1.reshape(1, -1), w2, b2.reshape(1, -1))


def _final(num, den, sk, b2):
    npad = num.shape[1]
    d = num.shape[2]

    def body(num_ref, den_ref, sk_ref, b2_ref, o_ref):
        nm = num_ref[0] + num_ref[1]
        dn = jnp.sum(den_ref[...], axis=0)
        o_ref[...] = nm / (dn[:, None] + 1e-16) + b2_ref[...] + sk_ref[...]

    return pl.pallas_call(
        body,
        grid=(npad // _BN,),
        in_specs=[pl.BlockSpec((_NC, _BN, d), lambda i: (0, i, 0)),
                  pl.BlockSpec((_NW, _BN), lambda i: (0, i)),
                  pl.BlockSpec((_BN, d), lambda i: (i, 0)),
                  pl.BlockSpec((1, d), lambda i: (0, 0))],
        out_specs=pl.BlockSpec((_BN, d), lambda i: (i, 0)),
        out_shape=jax.ShapeDtypeStruct((npad, d), jnp.float32),
    )(num, den, sk, b2.reshape(1, -1))


def _edge_agg(xl, xr, src, dst, att):
    npad, d = xl.shape
    e = src.shape[0]
    ew = e // _NW
    nch = ew // _CH
    rps = npad // _NS
    nj = d // 16

    mesh = plsc.VectorSubcoreMesh(core_axis_name="c", subcore_axis_name="s",
                                  num_cores=_NC, num_subcores=_NS)

    nd = 2
    ni = 8

    @functools.partial(
        pl.kernel,
        out_type=[jax.ShapeDtypeStruct((_NC, npad, d), jnp.float32),
                  jax.ShapeDtypeStruct((_NW * npad,), jnp.float32)],
        mesh=mesh,
        compiler_params=pltpu.CompilerParams(needs_layout_passes=False),
        scratch_types=[
            pltpu.VMEM((ni, _CH), jnp.int32),
            pltpu.VMEM((ni, _CH), jnp.int32),
            pltpu.VMEM((nd, _CH, d), jnp.float32),
            pltpu.VMEM((nd, _CH, d), jnp.float32),
            pltpu.VMEM((d,), jnp.float32),
            pltpu.VMEM((npad,), jnp.float32),
            pltpu.VMEM_SHARED((npad, d), jnp.float32),
            pltpu.SemaphoreType.DMA((ni,)),
            pltpu.SemaphoreType.DMA((ni,)),
            pltpu.SemaphoreType.DMA((nd,)),
            pltpu.SemaphoreType.DMA((nd,)),
            pltpu.SemaphoreType.DMA((nd,)),
        ],
    )
    def k(xl_h, xr_h, src_h, dst_h, att_h, num_h, den_h,
          srcm, dstm, xls, xrd, attv, denv, numsh,
          isem_s, isem_d, gsem_l, gsem_r, ssem):
        cid = lax.axis_index("c")
        sid = lax.axis_index("s")
        wid = cid * _NS + sid
        zero16 = jnp.zeros((16,), jnp.float32)
        base_e = wid * ew

        def idx_start(c, slot):
            off = base_e + c * _CH
            pltpu.async_copy(src_h.at[pl.ds(off, _CH)], srcm.at[slot],
                             isem_s.at[slot])
            pltpu.async_copy(dst_h.at[pl.ds(off, _CH)], dstm.at[slot],
                             isem_d.at[slot])

        def idx_wait(slot):
            pltpu.make_async_copy(src_h.at[pl.ds(0, _CH)], srcm.at[slot],
                                  isem_s.at[slot]).wait()
            pltpu.make_async_copy(dst_h.at[pl.ds(0, _CH)], dstm.at[slot],
                                  isem_d.at[slot]).wait()

        def gather_start(islot, s):
            pltpu.async_copy(xl_h.at[srcm.at[islot]], xls.at[s], gsem_l.at[s])
            pltpu.async_copy(xr_h.at[dstm.at[islot]], xrd.at[s], gsem_r.at[s])

        def gather_wait(s):
            pltpu.make_async_copy(xl_h.at[pl.ds(0, _CH)], xls.at[s],
                                  gsem_l.at[s]).wait()
            pltpu.make_async_copy(xr_h.at[pl.ds(0, _CH)], xrd.at[s],
                                  gsem_r.at[s]).wait()

        def scatter_wait(s):
            pltpu.make_async_copy(xl_h.at[pl.ds(0, _CH)], xls.at[s],
                                  ssem.at[s]).wait()

        def zrow(i, _):
            for j in range(nj):
                xls[0, i, pl.ds(16 * j, 16)] = zero16
            return 0
        lax.fori_loop(0, _CH, zrow, 0)

        def zden(i, _):
            denv[pl.ds(i * 16, 16)] = zero16
            return 0
        lax.fori_loop(0, npad // 16, zden, 0)

        def znum(i, _):
            pltpu.sync_copy(xls.at[0], numsh.at[pl.ds(sid * rps + i * _CH, _CH)])
            return 0
        lax.fori_loop(0, rps // _CH, znum, 0)

        pltpu.sync_copy(att_h, attv)

        for c in range(_PRE):
            idx_start(c, c)
        idx_wait(0)
        gather_start(0, 0)

        plsc.subcore_barrier()

        attr = [attv[pl.ds(16 * j, 16)] for j in range(nj)]
        lane0 = lax.iota(jnp.int32, 16) == 0

        def chunk(g, _):
            s = g % nd
            sn = (g + 1) % nd

            @pl.when(g + _PRE < nch)
            def _():
                idx_start(g + _PRE, (g + _PRE) % ni)

            @pl.when(g + 1 < nch)
            def _():
                @pl.when(g >= 1)
                def _():
                    scatter_wait(sn)
                idx_wait((g + 1) % ni)
                gather_start((g + 1) % ni, sn)

            gather_wait(s)

            ig = g % ni

            def group(gg, _):
                dvec = dstm[ig, pl.ds(gg * 16, 16)]
                for i in range(16):
                    ee = gg * 16 + i
                    lvs = []
                    acc = zero16
                    for j in range(nj):
                        lv = xls[s, ee, pl.ds(16 * j, 16)]
                        rv = xrd[s, ee, pl.ds(16 * j, 16)]
                        lvs.append(lv)
                        m = lv + rv
                        m = jnp.maximum(m, 0.2 * m)
                        acc = acc + m * attr[j]
                    ex = jnp.exp(jnp.full((16,), jnp.sum(acc), jnp.float32))
                    for j in range(nj):
                        xls[s, ee, pl.ds(16 * j, 16)] = ex * lvs[j]
                    plsc.addupdate_scatter(denv,
                                           [jnp.full((16,), dvec[i], jnp.int32)],
                                           ex, mask=lane0)
                return 0
            lax.fori_loop(0, _CH // 16, group, 0)

            pltpu.async_copy(xls.at[s], numsh.at[dstm.at[ig]], ssem.at[s],
                             add=True)
            return 0
        lax.fori_loop(0, nch, chunk, 0)

        scatter_wait((nch - 2) % nd)
        scatter_wait((nch - 1) % nd)

        plsc.subcore_barrier()
        pltpu.sync_copy(denv, den_h.at[pl.ds(wid * npad, npad)])

        def wout(i, _):
            r0 = sid * rps + i * _CH
            pltpu.sync_copy(numsh.at[pl.ds(r0, _CH)], num_h.at[cid, pl.ds(r0, _CH)])
            return 0
        lax.fori_loop(0, rps // _CH, wout, 0)

    num, den_flat = k(xl, xr, src, dst, att)
    return num, den_flat.reshape(_NW, npad)


def kernel(x, edge_index, edge_attr, Wl1, bl1, Wr1, br1, att1, bias1, Wlin1, blin1,
           Wl2, bl2, Wr2, br2, att2, bias2, Wlin2, blin2):
    n = x.shape[0]
    npad = ((n + _BN - 1) // _BN) * _BN
    xp = jnp.pad(x, ((0, npad - n), (0, 0)))

    e = edge_index.shape[1]
    ep = ((e + _NW * _CH - 1) // (_NW * _CH)) * (_NW * _CH)
    src = jnp.pad(edge_index[0], (0, ep - e), constant_values=npad - 1)
    dst = jnp.pad(edge_index[1], (0, ep - e), constant_values=npad - 1)
    xl1, xr1, sk1 = _mm3(xp, Wl1, bl1, Wr1, br1, Wlin1, blin1)
    num1, den1 = _edge_agg(xl1, xr1, src, dst, att1)
    xl2, xr2, sk2 = _mid(num1, den1, sk1, bias1, Wl2, bl2, Wr2, br2, Wlin2, blin2)
    num2, den2 = _edge_agg(xl2, xr2, src, dst, att2)
    outp = _final(num2, den2, sk2, bias2)
    return outp[:n]
